# baseline (speedup 1.0000x reference)
import functools

import jax
import jax.numpy as jnp
from jax import lax
from jax.experimental import pallas as pl
from jax.experimental.pallas import tpu as pltpu
from jax.experimental.pallas import tpu_sc as plsc


def kernel(x, emb_table, pos_table):
    B, S = x.shape
    V, D = emb_table.shape
    scale = float(D) ** 0.5
    L = 16
    CH = 128
    NCH = B // CH

    info = plsc.get_sparse_core_info()
    NC, NS = info.num_cores, info.num_subcores
    NW = NC * NS
    max_slabs = -(-S // NW)

    xT = jnp.transpose(x)
    tab128 = jnp.pad(emb_table, ((0, 0), (0, D)))
    pos = pos_table[:S]

    mesh = plsc.VectorSubcoreMesh(core_axis_name="c", subcore_axis_name="s")

    @functools.partial(
        pl.kernel,
        mesh=mesh,
        compiler_params=pltpu.CompilerParams(
            use_tc_tiling_on_sc=False, needs_layout_passes=False
        ),
        out_type=jax.ShapeDtypeStruct((S, 8, 8, 8, CH), jnp.float32),
        scratch_types=[
            pltpu.VMEM((B,), jnp.int32),
            pltpu.VMEM((2, CH, 2 * D), jnp.float32),
            pltpu.VMEM((8, 8, 8, CH), jnp.float32),
            pltpu.VMEM((S, D), jnp.float32),
            pltpu.SemaphoreType.DMA,
            pltpu.SemaphoreType.DMA((2,)),
            pltpu.SemaphoreType.DMA,
        ],
    )
    def emb_kernel(xT_hbm, tab_hbm, pos_hbm, out_hbm,
                   idx_v, gbuf, slab, pos_v, isem, gsem, wsem):
        wid = lax.axis_index("s") * NC + lax.axis_index("c")
        pltpu.sync_copy(pos_hbm, pos_v)
        iota = lax.iota(jnp.int32, L)

        def start_gather(c):
            return pltpu.async_copy(
                tab_hbm.at[idx_v.at[pl.ds(c * CH, CH)]],
                gbuf.at[c % 2], gsem.at[c % 2])

        def compute_chunk(c, s_dyn):
            gb = c % 2

            def f_body(f, carry):
                prow = jnp.full((L,), carry, dtype=jnp.int32)
                fcol = jnp.full((L,), f, dtype=jnp.int32)
                pvec = plsc.load_gather(pos_v, [prow, fcol])
                a = f // 8
                u = f % 8
                for g in range(CH // L):
                    rows = iota + (g * L)
                    vals = plsc.load_gather(gbuf.at[gb], [rows, fcol])
                    slab[a, c, u, pl.ds(g * L, L)] = vals * scale + pvec
                return carry

            lax.fori_loop(0, D, f_body, s_dyn)

        for k in range(max_slabs):
            s_dyn = wid + k * NW

            @pl.when(s_dyn < S)
            def _():
                pltpu.sync_copy(xT_hbm.at[s_dyn], idx_v)
                handles = [start_gather(0)]
                if k > 0:
                    pltpu.make_async_copy(
                        slab, out_hbm.at[s_dyn - NW], wsem).wait()
                for c in range(NCH):
                    if c + 1 < NCH:
                        handles.append(start_gather(c + 1))
                    handles[c].wait()
                    compute_chunk(c, s_dyn)
                pltpu.async_copy(slab, out_hbm.at[s_dyn], wsem)

        pltpu.make_async_copy(slab, out_hbm.at[wid], wsem).wait()

    out5 = emb_kernel(xT, tab128, pos)
    return jnp.transpose(out5, (2, 4, 0, 1, 3)).reshape(B, S, D)


# speedup vs baseline: 1.0771x; 1.0771x over previous
import functools

import jax
import jax.numpy as jnp
from jax import lax
from jax.experimental import pallas as pl
from jax.experimental.pallas import tpu as pltpu
from jax.experimental.pallas import tpu_sc as plsc


def kernel(x, emb_table, pos_table):
    B, S = x.shape
    V, D = emb_table.shape
    scale = float(D) ** 0.5
    L = 16
    CH = 128
    NCH = B // CH

    info = plsc.get_sparse_core_info()
    NC, NS = info.num_cores, info.num_subcores
    NW = NC * NS
    max_slabs = -(-S // NW)

    xT = jnp.transpose(x)
    tab128 = jnp.pad(emb_table, ((0, 0), (0, D)))
    pos = pos_table[:S]

    mesh = plsc.VectorSubcoreMesh(core_axis_name="c", subcore_axis_name="s")

    @functools.partial(
        pl.kernel,
        mesh=mesh,
        compiler_params=pltpu.CompilerParams(
            use_tc_tiling_on_sc=False, needs_layout_passes=False
        ),
        out_type=jax.ShapeDtypeStruct((S, 8 * 8 * 8 * CH), jnp.float32),
        scratch_types=[
            pltpu.VMEM((B,), jnp.int32),
            pltpu.VMEM((2, CH, 2 * D), jnp.float32),
            pltpu.VMEM((8 * 8 * 8 * CH,), jnp.float32),
            pltpu.VMEM((S, D), jnp.float32),
            pltpu.SemaphoreType.DMA,
            pltpu.SemaphoreType.DMA((2,)),
            pltpu.SemaphoreType.DMA,
        ],
    )
    def emb_kernel(xT_hbm, tab_hbm, pos_hbm, out_hbm,
                   idx_v, gbuf, slab, pos_v, isem, gsem, wsem):
        wid = lax.axis_index("s") * NC + lax.axis_index("c")
        pltpu.sync_copy(pos_hbm, pos_v)
        iota = lax.iota(jnp.int32, L)
        base_v = (iota // 8) * 8192 + (iota % 8) * 128
        TU = 4

        def start_gather(c):
            return pltpu.async_copy(
                tab_hbm.at[idx_v.at[pl.ds(c * CH, CH)]],
                gbuf.at[c % 2], gsem.at[c % 2])

        def compute_chunk(c, pos_regs):
            gb = c % 2
            av = [base_v + (j * 2 * 8192 + c * CH * 8) for j in range(4)]

            def tok_body(i, carry):
                for t in range(TU):
                    tok = i * TU + t
                    for j in range(4):
                        g = gbuf[gb, tok, pl.ds(j * L, L)]
                        plsc.store_scatter(
                            slab, [av[j] + tok], g * scale + pos_regs[j])
                return carry

            lax.fori_loop(0, CH // TU, tok_body, 0)

        def slab_body(k, carry):
            s_dyn = wid + k * NW

            @pl.when(s_dyn < S)
            def _():
                pltpu.sync_copy(xT_hbm.at[s_dyn], idx_v)
                handles = [start_gather(0)]
                pos_regs = [pos_v[s_dyn, pl.ds(j * L, L)] for j in range(4)]

                @pl.when(k > 0)
                def _():
                    pltpu.make_async_copy(
                        slab, out_hbm.at[s_dyn - NW], wsem).wait()

                for c in range(NCH):
                    if c + 1 < NCH:
                        handles.append(start_gather(c + 1))
                    handles[c].wait()
                    compute_chunk(c, pos_regs)
                pltpu.async_copy(slab, out_hbm.at[s_dyn], wsem)

            return carry

        lax.fori_loop(0, max_slabs, slab_body, 0)

        pltpu.make_async_copy(slab, out_hbm.at[wid], wsem).wait()

    out5 = emb_kernel(xT, tab128, pos).reshape(S, 8, 8, 8, CH)
    return jnp.transpose(out5, (2, 4, 0, 1, 3)).reshape(B, S, D)


# speedup vs baseline: 1.2427x; 1.1538x over previous
import functools

import jax
import jax.numpy as jnp
from jax import lax
from jax.experimental import pallas as pl
from jax.experimental.pallas import tpu as pltpu
from jax.experimental.pallas import tpu_sc as plsc


def kernel(x, emb_table, pos_table):
    B, S = x.shape
    V, D = emb_table.shape
    scale = float(D) ** 0.5
    NVEC = D // 16

    info = plsc.get_sparse_core_info()
    NC, NS = info.num_cores, info.num_subcores
    NW = NC * NS
    seqs_per_w = B // NW

    C0 = 104
    C1 = S - C0

    NB = 4
    RU = 4

    pos = pos_table[:S]

    mesh = plsc.VectorSubcoreMesh(core_axis_name="c", subcore_axis_name="s")

    @functools.partial(
        pl.kernel,
        mesh=mesh,
        compiler_params=pltpu.CompilerParams(use_tc_tiling_on_sc=False),
        out_type=jax.ShapeDtypeStruct((B, S, D), jnp.float32),
        scratch_types=[
            pltpu.VMEM((NB, S), jnp.int32),
            pltpu.VMEM((NB, S, D), jnp.float32),
            pltpu.VMEM((S, D), jnp.float32),
            pltpu.SemaphoreType.DMA((NB,)),
            pltpu.SemaphoreType.DMA((NB,)),
        ],
    )
    def emb_kernel(x_hbm, tab_hbm, pos_hbm, out_hbm, idx_v, rows_v, pos_v,
                   gsem, wsem):
        wid = lax.axis_index("s") * NC + lax.axis_index("c")
        base = wid * seqs_per_w
        pltpu.sync_copy(pos_hbm, pos_v)

        def start_fetch(j):
            b = j % NB
            pltpu.sync_copy(x_hbm.at[base + j], idx_v.at[b])
            g0 = pltpu.async_copy(
                tab_hbm.at[idx_v.at[b, pl.ds(0, C0)]],
                rows_v.at[b, pl.ds(0, C0)],
                gsem.at[b],
            )
            g1 = pltpu.async_copy(
                tab_hbm.at[idx_v.at[b, pl.ds(C0, C1)]],
                rows_v.at[b, pl.ds(C0, C1)],
                gsem.at[b],
            )
            return (g0, g1)

        def compute(b):
            def body(i, carry):
                r = i * RU
                for rr in range(RU):
                    for c in range(NVEC):
                        sl = pl.ds(c * 16, 16)
                        rows_v[b, r + rr, sl] = (
                            rows_v[b, r + rr, sl] * scale + pos_v[r + rr, sl]
                        )
                return carry

            lax.fori_loop(0, S // RU, body, 0)

        gh = [None] * NB
        wh = [None] * NB
        gh[0] = start_fetch(0)
        gh[1] = start_fetch(1)
        for j in range(seqs_per_w):
            b = j % NB
            f = j + 2
            if f < seqs_per_w:
                fb = f % NB
                if wh[fb] is not None:
                    wh[fb].wait()
                gh[fb] = start_fetch(f)
            gh[b][0].wait()
            gh[b][1].wait()
            compute(b)
            wh[b] = pltpu.async_copy(rows_v.at[b], out_hbm.at[base + j],
                                     wsem.at[b])
        for b in range(NB):
            if wh[b] is not None:
                wh[b].wait()

    return emb_kernel(x, emb_table, pos)
